# baseline (speedup 1.0000x reference)
import functools

import jax
import jax.numpy as jnp
from jax.experimental import pallas as pl
from jax.experimental.pallas import tpu as pltpu

S, D, E, F, K = 2048, 1024, 8, 2048, 2
T = 256
NB = (S * K) // T + E
R = NB * T


def _router_body(x_ref, wr_ref, idx_ref, gate_ref):
    x = x_ref[...]
    logits = jnp.dot(x, wr_ref[...], preferred_element_type=jnp.float32)
    lane = jax.lax.broadcasted_iota(jnp.int32, (S, E), 1)
    m1 = jnp.max(logits, axis=1, keepdims=True)
    i1 = jnp.argmax(logits, axis=1)[:, None]
    masked = jnp.where(lane == i1, -jnp.inf, logits)
    m2 = jnp.max(masked, axis=1, keepdims=True)
    i2 = jnp.argmax(masked, axis=1)[:, None]
    d = jnp.exp(m2 - m1)
    g1 = 1.0 / (1.0 + d)
    g2 = d / (1.0 + d)
    idx_ref[...] = jnp.concatenate([i1, i2], axis=1).astype(jnp.int32)
    gate_ref[...] = jnp.concatenate([g1, g2], axis=1)


def _router(x2d, wr):
    return pl.pallas_call(
        _router_body,
        out_shape=(
            jax.ShapeDtypeStruct((S, K), jnp.int32),
            jax.ShapeDtypeStruct((S, K), jnp.float32),
        ),
    )(x2d, wr)


def _ffn_body(be_ref, x_ref, w1_ref, b1_ref, w2_ref, b2_ref, o_ref):
    del be_ref
    h = jnp.dot(x_ref[...], w1_ref[0], preferred_element_type=jnp.float32)
    h = jax.nn.gelu(h + b1_ref[0])
    y = jnp.dot(h, w2_ref[0], preferred_element_type=jnp.float32)
    o_ref[...] = y + b2_ref[0]


def _ffn(x_sorted, W1, b1, W2, b2, block_expert):
    grid_spec = pltpu.PrefetchScalarGridSpec(
        num_scalar_prefetch=1,
        grid=(NB,),
        in_specs=[
            pl.BlockSpec((T, D), lambda b, be: (b, 0)),
            pl.BlockSpec((1, D, F), lambda b, be: (be[b], 0, 0)),
            pl.BlockSpec((1, 1, F), lambda b, be: (be[b], 0, 0)),
            pl.BlockSpec((1, F, D), lambda b, be: (be[b], 0, 0)),
            pl.BlockSpec((1, 1, D), lambda b, be: (be[b], 0, 0)),
        ],
        out_specs=pl.BlockSpec((T, D), lambda b, be: (b, 0)),
    )
    return pl.pallas_call(
        _ffn_body,
        grid_spec=grid_spec,
        out_shape=jax.ShapeDtypeStruct((R, D), jnp.float32),
    )(block_expert, x_sorted, W1, b1.reshape(E, 1, F), W2, b2.reshape(E, 1, D))


def kernel(x, W_router, W1, b1, W2, b2):
    x2d = x.reshape(S, D)
    idx, gates = _router(x2d, W_router)

    pairs_e = idx.reshape(S * K)
    onehot = (pairs_e[:, None] == jnp.arange(E, dtype=jnp.int32)[None, :])
    cum = jnp.cumsum(onehot.astype(jnp.int32), axis=0)
    counts = cum[-1]
    nblk = (counts + T - 1) // T
    blk_base = jnp.concatenate(
        [jnp.zeros((1,), jnp.int32), jnp.cumsum(nblk)[:-1].astype(jnp.int32)])
    rank = jnp.take_along_axis(cum, pairs_e[:, None], axis=1)[:, 0] - 1
    pos = blk_base[pairs_e] * T + rank
    bids = jnp.arange(NB, dtype=jnp.int32)
    block_expert = (
        jnp.sum((blk_base[None, :] <= bids[:, None]).astype(jnp.int32), axis=1)
        - 1).astype(jnp.int32)

    tok = jnp.arange(S * K, dtype=jnp.int32) // K
    x_sorted = jnp.zeros((R, D), jnp.float32).at[pos].set(x2d[tok])

    y_sorted = _ffn(x_sorted, W1, b1, W2, b2, block_expert)

    y2 = y_sorted[pos].reshape(S, K, D)
    out = jnp.sum(gates[:, :, None] * y2, axis=1)
    return out.reshape(1, S, D)
